# baseline (speedup 1.0000x reference)
import functools

import jax
import jax.numpy as jnp
from jax import lax
from jax.experimental import pallas as pl
from jax.experimental.pallas import tpu as pltpu
from jax.experimental.pallas import tpu_sc as plsc

PADDING_IDX = 1
LANES = 16
CHUNK = 64


def _make_sc_kernel(bsz, seq_len, n_table, dim, n_workers, n_cores):
    s_per_w = seq_len // n_workers
    mesh = plsc.VectorSubcoreMesh(core_axis_name="c", subcore_axis_name="s")

    @functools.partial(
        pl.kernel,
        mesh=mesh,
        out_type=jax.ShapeDtypeStruct((bsz, seq_len, dim), jnp.float32),
        scratch_types=[
            pltpu.VMEM((bsz, s_per_w), jnp.int32),
            pltpu.VMEM((bsz, s_per_w), jnp.int32),
            pltpu.VMEM((CHUNK, dim), jnp.float32),
            pltpu.SemaphoreType.DMA,
        ],
    )
    def sc_posemb(inp_hbm, w_hbm, out_hbm, inp_v, pos_v, rows_v, sem):
        wid = lax.axis_index("s") * n_cores + lax.axis_index("c")
        base_s = wid * s_per_w
        for b in range(bsz):
            pltpu.sync_copy(inp_hbm.at[b, pl.ds(base_s, s_per_w)], inp_v.at[b])
        zero = jnp.zeros((LANES,), jnp.int32)
        one = jnp.full((LANES,), 1, jnp.int32)
        pad = jnp.full((LANES,), PADDING_IDX, jnp.int32)
        for j in range(s_per_w // LANES):
            sl = pl.ds(j * LANES, LANES)
            cum = zero
            for b in range(bsz):
                m = jnp.where(inp_v[b, sl] == pad, zero, one)
                cum = cum + m
                pos_v[b, sl] = cum * m + pad
        for b in range(bsz):
            for c in range(s_per_w // CHUNK):
                idx = pos_v.at[b, pl.ds(c * CHUNK, CHUNK)]
                pltpu.async_copy(w_hbm.at[idx], rows_v, sem).wait()
                pltpu.sync_copy(
                    rows_v, out_hbm.at[b, pl.ds(base_s + c * CHUNK, CHUNK)]
                )

    return sc_posemb


def kernel(input, weights):
    bsz, seq_len = input.shape
    n_table, dim = weights.shape
    info = plsc.get_sparse_core_info()
    n_workers = info.num_cores * info.num_subcores
    sc = _make_sc_kernel(bsz, seq_len, n_table, dim, n_workers, info.num_cores)
    return sc(input, weights)
